# baseline (speedup 1.0000x reference)
import functools

import jax
import jax.numpy as jnp
from jax import lax
from jax.experimental import pallas as pl
from jax.experimental.pallas import tpu as pltpu
from jax.experimental.pallas import tpu_sc as plsc

N_NODES = 50000
D = 128
L = 16
NT = 16
NSC = 2
N_PASS = 3
CS = 8336
GR = 8400
CPAD = 8448
RSH = CPAD // NT
ECH = 1600
VPC = ECH // L
NB = 128
LCAP = 2 * ECH + 256


def _sc_body(nchunks, tile_share,
             xu, xi, s0, d0, s1, d1, out0, cnt0, out1, cnt1,
             lst, e_s, e_d, rows, ones, srcb, didx, zc,
             acc, cnt, gsem, esem, ssem, csem):
    cid = lax.axis_index("c")
    sid = lax.axis_index("s")

    def _fill(r, _):
        ones[r, :] = jnp.ones((L,), jnp.float32)
        zc[r, :] = jnp.zeros((L,), jnp.float32)
        return 0
    lax.fori_loop(0, NB, _fill, 0)

    for rel in range(2):
        xh = xu if rel == 0 else xi
        sh = s0 if rel == 0 else s1
        dh = d0 if rel == 0 else d1
        oh = out0 if rel == 0 else out1
        ch = cnt0 if rel == 0 else cnt1
        for p in range(N_PASS):
            lo = ((p * NSC + cid) * CS).astype(jnp.int32)
            size = jnp.minimum(N_NODES - lo, CS).astype(jnp.int32)
            hi = lo + size

            def _zrow(r, _):
                for k in range(D // L):
                    rows[0, r, pl.ds(k * L, L)] = jnp.zeros((L,), jnp.float32)
                return 0
            lax.fori_loop(0, NB, _zrow, 0)
            r0 = sid * RSH
            for cch in range(RSH // NB):
                pltpu.sync_copy(rows.at[0], acc.at[pl.ds(r0 + cch * NB, NB)])
                pltpu.sync_copy(zc, cnt.at[pl.ds(r0 + cch * NB, NB)])
            tail = RSH % NB
            if tail:
                base_t = r0 + (RSH // NB) * NB
                pltpu.sync_copy(rows.at[0, pl.ds(0, tail)], acc.at[pl.ds(base_t, tail)])
                pltpu.sync_copy(zc.at[pl.ds(0, tail)], cnt.at[pl.ds(base_t, tail)])
            plsc.subcore_barrier()

            def _fire(j, b):
                base = b * NB
                for k in range(NB // L):
                    pk = lst[pl.ds(base + k * L, L)]
                    didx[j, pl.ds(k * L, L)] = lax.shift_right_logical(pk, 16)
                    srcb[j, pl.ds(k * L, L)] = pk & 0xFFFF
                pltpu.async_copy(xh.at[srcb.at[j]], rows.at[j], gsem.at[j])

            def _waitg(j, b):
                pltpu.make_async_copy(xh.at[srcb.at[j]],
                                      rows.at[j], gsem.at[j]).wait()

            def _scat(j):
                pltpu.async_copy(rows.at[j], acc.at[didx.at[j]],
                                 ssem.at[j], add=True)
                pltpu.async_copy(ones, cnt.at[didx.at[j]],
                                 csem.at[j], add=True)

            def _wait_scat(j):
                pltpu.make_async_copy(rows.at[j], acc.at[didx.at[j]],
                                      ssem.at[j]).wait()
                pltpu.make_async_copy(ones, cnt.at[didx.at[j]],
                                      csem.at[j]).wait()

            def _drain(nb):
                @pl.when(nb > 0)
                def _p():
                    _fire(0, 0)

                def _pair(i, _):
                    b = i * 2

                    @pl.when((b + 1 < nb) & (b >= 2))
                    def _ws1():
                        _wait_scat(1)

                    @pl.when(b + 1 < nb)
                    def _f1():
                        _fire(1, b + 1)
                    _waitg(0, b)
                    _scat(0)

                    @pl.when(b + 2 < nb)
                    def _f0():
                        _wait_scat(0)
                        _fire(0, b + 2)

                    @pl.when(b + 1 < nb)
                    def _w1():
                        _waitg(1, b + 1)
                        _scat(1)
                    return 0
                lax.fori_loop(0, (nb + 1) // 2, _pair, 0)

                @pl.when(nb > 0)
                def _e0():
                    _wait_scat(0)

                @pl.when(nb > 1)
                def _e1():
                    _wait_scat(1)

            def _fire_e(j, c):
                ebase = sid * tile_share + c * ECH
                pltpu.async_copy(sh.at[pl.ds(ebase, ECH)], e_s.at[j],
                                 esem.at[j])
                pltpu.async_copy(dh.at[pl.ds(ebase, ECH)], e_d.at[j],
                                 esem.at[j])

            def _wait_e(j, c):
                ebase = sid * tile_share + c * ECH
                pltpu.make_async_copy(sh.at[pl.ds(ebase, ECH)], e_s.at[j],
                                      esem.at[j]).wait()
                pltpu.make_async_copy(dh.at[pl.ds(ebase, ECH)], e_d.at[j],
                                      esem.at[j]).wait()

            def _scan(j, nm):
                def _vreg(v, nm2):
                    sv = e_s[j, pl.ds(v * L, L)]
                    dv = e_d[j, pl.ds(v * L, L)]
                    doff = dv - lo
                    m = (plsc.bitcast(doff, jnp.uint32)
                         < plsc.bitcast(jnp.full((L,), size, jnp.int32),
                                        jnp.uint32))
                    packed = sv | lax.shift_left(doff, 16)
                    plsc.store_compressed(lst.at[pl.ds(nm2, L)], packed,
                                          mask=m)
                    pc = plsc.all_reduce_population_count(m)
                    return nm2 + pc[0]
                return lax.fori_loop(0, VPC, _vreg, nm)

            ngroups = nchunks // 2
            _fire_e(0, 0)

            def _grp(g, nm):
                c0 = g * 2
                _fire_e(1, c0 + 1)
                _wait_e(0, c0)
                nm = _scan(0, nm)

                @pl.when(g < ngroups - 1)
                def _pf():
                    _fire_e(0, c0 + 2)
                _wait_e(1, c0 + 1)
                nm = _scan(1, nm)

                nb = nm // NB
                _drain(nb)
                rebase = nb * NB
                for k in range(NB // L):
                    pkv = lst[pl.ds(rebase + k * L, L)]
                    lst[pl.ds(k * L, L)] = pkv
                return nm - rebase
            nm = lax.fori_loop(0, ngroups, _grp, jnp.int32(0))

            for k in range(NB // L):
                lst[pl.ds(nm + k * L, L)] = jnp.full((L,), GR << 16, jnp.int32)
            _drain((nm + (NB - 1)) // NB)
            plsc.subcore_barrier()

            nrows = jnp.clip(size - (sid * RSH).astype(jnp.int32), 0, RSH)
            nfull = nrows // NB

            def _wblock(rbase, m):
                pltpu.sync_copy(acc.at[pl.ds(rbase, m)],
                                oh.at[pl.ds(lo + rbase, m)])
                pltpu.sync_copy(cnt.at[pl.ds(rbase, m)],
                                ch.at[pl.ds(lo + rbase, m)])

            def _wchunk(cch, _):
                _wblock(r0 + cch * NB, NB)
                return 0
            lax.fori_loop(0, nfull, _wchunk, 0)

            def _wtail(g, _):
                _wblock(r0 + nfull * NB + g * 8, 8)
                return 0
            lax.fori_loop(0, (nrows - nfull * NB) // 8, _wtail, 0)
            plsc.subcore_barrier()


def _sc_aggregate(x_user, x_item, s_click, d_click, s_cb, d_cb):
    ep = s_click.shape[0]
    tile_share = ep // NT
    nchunks = tile_share // ECH
    mesh = plsc.VectorSubcoreMesh(core_axis_name="c", subcore_axis_name="s")
    f32 = jnp.float32
    kern = pl.kernel(
        functools.partial(_sc_body, nchunks, tile_share),
        out_type=(jax.ShapeDtypeStruct((N_NODES, D), f32),
                  jax.ShapeDtypeStruct((N_NODES, L), f32),
                  jax.ShapeDtypeStruct((N_NODES, D), f32),
                  jax.ShapeDtypeStruct((N_NODES, L), f32)),
        mesh=mesh,
        compiler_params=pltpu.CompilerParams(needs_layout_passes=False,
                                             use_tc_tiling_on_sc=False),
        scratch_types=[
            pltpu.VMEM((LCAP,), jnp.int32),
            pltpu.VMEM((2, ECH), jnp.int32),
            pltpu.VMEM((2, ECH), jnp.int32),
            pltpu.VMEM((2, NB, D), f32),
            pltpu.VMEM((NB, L), f32),
            pltpu.VMEM((2, NB), jnp.int32),
            pltpu.VMEM((2, NB), jnp.int32),
            pltpu.VMEM((NB, L), f32),
            pltpu.VMEM_SHARED((CPAD, D), f32),
            pltpu.VMEM_SHARED((CPAD, L), f32),
            pltpu.SemaphoreType.DMA((2,)),
            pltpu.SemaphoreType.DMA((2,)),
            pltpu.SemaphoreType.DMA((2,)),
            pltpu.SemaphoreType.DMA((2,)),
        ],
    )
    return kern(x_user, x_item, s_click, d_click, s_cb, d_cb)


def _mm_body(x_ref, c_ref, w_ref, b_ref, o_ref):
    recip = 1.0 / jnp.maximum(c_ref[...][:, 0:1], 1.0)
    o_ref[...] = jnp.dot(x_ref[...] * recip, w_ref[...],
                         preferred_element_type=jnp.float32) + b_ref[...]


def _transform(sum_x, cnt_x, W, b):
    bm = 2000
    grid = (sum_x.shape[0] // bm,)
    return pl.pallas_call(
        _mm_body,
        grid=grid,
        in_specs=[
            pl.BlockSpec((bm, D), lambda i: (i, 0)),
            pl.BlockSpec((bm, L), lambda i: (i, 0)),
            pl.BlockSpec((D, D), lambda i: (0, 0)),
            pl.BlockSpec((1, D), lambda i: (0, 0)),
        ],
        out_specs=pl.BlockSpec((bm, D), lambda i: (i, 0)),
        out_shape=jax.ShapeDtypeStruct((sum_x.shape[0], D), jnp.float32),
    )(sum_x, cnt_x, W, b.reshape(1, D))


def kernel(x_user, x_item, edge_index_click, edge_index_clicked_by,
           W_click, b_click, W_clicked_by, b_clicked_by):
    e = edge_index_click.shape[1]
    ep = ((e + NT * ECH - 1) // (NT * ECH)) * (NT * ECH)
    pad = ep - e
    sentinel = jnp.int32(1 << 28)

    def prep(ei):
        s = ei[0].astype(jnp.int32)
        d = ei[1].astype(jnp.int32)
        if pad:
            s = jnp.concatenate([s, jnp.zeros((pad,), jnp.int32)])
            d = jnp.concatenate([d, jnp.full((pad,), sentinel, jnp.int32)])
        return s, d

    s_click, d_click = prep(edge_index_click)
    s_cb, d_cb = prep(edge_index_clicked_by)

    sum_click, cnt_click, sum_cb, cnt_cb = _sc_aggregate(
        x_user, x_item, s_click, d_click, s_cb, d_cb)
    h_item = _transform(sum_click, cnt_click, W_click, b_click)
    h_user = _transform(sum_cb, cnt_cb, W_clicked_by, b_clicked_by)
    return (h_user, h_item)


# speedup vs baseline: 1.0215x; 1.0215x over previous
import functools

import jax
import jax.numpy as jnp
from jax import lax
from jax.experimental import pallas as pl
from jax.experimental.pallas import tpu as pltpu
from jax.experimental.pallas import tpu_sc as plsc

N_NODES = 50000
D = 128
L = 16
NT = 16
NSC = 2
N_PASS = 3
CS = 8336
GR = 8400
CPAD = 8448
RSH = CPAD // NT
ECH = 1600
VPC = ECH // L
NB = 128
LCAP = 2 * ECH + 256


def _sc_body(nchunks, tile_share,
             xu, xi, s0, d0, s1, d1, out0, cnt0, out1, cnt1,
             lst, e_s, e_d, rows, ones, srcb, didx, zc,
             acc, cnt, gsem, esem, ssem, csem):
    cid = lax.axis_index("c")
    sid = lax.axis_index("s")

    def _fill(r, _):
        ones[r, :] = jnp.ones((L,), jnp.float32)
        zc[r, :] = jnp.zeros((L,), jnp.float32)
        return 0
    lax.fori_loop(0, NB, _fill, 0)

    for rel in range(2):
        xh = xu if rel == 0 else xi
        sh = s0 if rel == 0 else s1
        dh = d0 if rel == 0 else d1
        oh = out0 if rel == 0 else out1
        ch = cnt0 if rel == 0 else cnt1
        for p in range(N_PASS):
            lo = ((p * NSC + cid) * CS).astype(jnp.int32)
            size = jnp.minimum(N_NODES - lo, CS).astype(jnp.int32)
            hi = lo + size

            def _zrow(r, _):
                for k in range(D // L):
                    rows[0, r, pl.ds(k * L, L)] = jnp.zeros((L,), jnp.float32)
                return 0
            lax.fori_loop(0, NB, _zrow, 0)
            r0 = sid * RSH
            for cch in range(RSH // NB):
                pltpu.sync_copy(rows.at[0], acc.at[pl.ds(r0 + cch * NB, NB)])
                pltpu.sync_copy(zc, cnt.at[pl.ds(r0 + cch * NB, NB)])
            tail = RSH % NB
            if tail:
                base_t = r0 + (RSH // NB) * NB
                pltpu.sync_copy(rows.at[0, pl.ds(0, tail)], acc.at[pl.ds(base_t, tail)])
                pltpu.sync_copy(zc.at[pl.ds(0, tail)], cnt.at[pl.ds(base_t, tail)])
            plsc.subcore_barrier()

            def _fire(j, b):
                base = b * NB
                for k in range(NB // L):
                    pk = lst[pl.ds(base + k * L, L)]
                    didx[j, pl.ds(k * L, L)] = lax.shift_right_logical(pk, 16)
                    srcb[j, pl.ds(k * L, L)] = pk & 0xFFFF
                pltpu.async_copy(xh.at[srcb.at[j]], rows.at[j], gsem.at[j])

            def _waitg(j, b):
                pltpu.make_async_copy(xh.at[srcb.at[j]],
                                      rows.at[j], gsem.at[j]).wait()

            def _scat(j):
                pltpu.async_copy(rows.at[j], acc.at[didx.at[j]],
                                 ssem.at[j], add=True)
                pltpu.async_copy(ones, cnt.at[didx.at[j]],
                                 csem.at[j], add=True)

            def _wait_scat(j):
                pltpu.make_async_copy(rows.at[j], acc.at[didx.at[j]],
                                      ssem.at[j]).wait()
                pltpu.make_async_copy(ones, cnt.at[didx.at[j]],
                                      csem.at[j]).wait()

            def _drain(nb):
                @pl.when(nb > 0)
                def _p():
                    _fire(0, 0)

                def _pair(i, _):
                    b = i * 2

                    @pl.when((b + 1 < nb) & (b >= 2))
                    def _ws1():
                        _wait_scat(1)

                    @pl.when(b + 1 < nb)
                    def _f1():
                        _fire(1, b + 1)
                    _waitg(0, b)
                    _scat(0)

                    @pl.when(b + 2 < nb)
                    def _f0():
                        _wait_scat(0)
                        _fire(0, b + 2)

                    @pl.when(b + 1 < nb)
                    def _w1():
                        _waitg(1, b + 1)
                        _scat(1)
                    return 0
                lax.fori_loop(0, (nb + 1) // 2, _pair, 0)

                @pl.when(nb > 0)
                def _e0():
                    _wait_scat(0)

                @pl.when(nb > 1)
                def _e1():
                    _wait_scat(1)

            def _fire_e(j, c):
                ebase = sid * tile_share + c * ECH
                pltpu.async_copy(sh.at[pl.ds(ebase, ECH)], e_s.at[j],
                                 esem.at[j])
                pltpu.async_copy(dh.at[pl.ds(ebase, ECH)], e_d.at[j],
                                 esem.at[j])

            def _wait_e(j, c):
                ebase = sid * tile_share + c * ECH
                pltpu.make_async_copy(sh.at[pl.ds(ebase, ECH)], e_s.at[j],
                                      esem.at[j]).wait()
                pltpu.make_async_copy(dh.at[pl.ds(ebase, ECH)], e_d.at[j],
                                      esem.at[j]).wait()

            def _scan(j, nm):
                def _vreg4(v4, nm2):
                    for u in range(4):
                        v = v4 * 4 + u
                        sv = e_s[j, pl.ds(v * L, L)]
                        dv = e_d[j, pl.ds(v * L, L)]
                        doff = dv - lo
                        m = (plsc.bitcast(doff, jnp.uint32)
                             < plsc.bitcast(jnp.full((L,), size, jnp.int32),
                                            jnp.uint32))
                        packed = sv | lax.shift_left(doff, 16)
                        plsc.store_compressed(lst.at[pl.ds(nm2, L)], packed,
                                              mask=m)
                        pc = plsc.all_reduce_population_count(m)
                        nm2 = nm2 + pc[0]
                    return nm2
                return lax.fori_loop(0, VPC // 4, _vreg4, nm)

            ngroups = nchunks // 2
            _fire_e(0, 0)

            def _grp(g, nm):
                c0 = g * 2
                _fire_e(1, c0 + 1)
                _wait_e(0, c0)
                nm = _scan(0, nm)

                @pl.when(g < ngroups - 1)
                def _pf():
                    _fire_e(0, c0 + 2)
                _wait_e(1, c0 + 1)
                nm = _scan(1, nm)

                nb = nm // NB
                _drain(nb)
                rebase = nb * NB
                for k in range(NB // L):
                    pkv = lst[pl.ds(rebase + k * L, L)]
                    lst[pl.ds(k * L, L)] = pkv
                return nm - rebase
            nm = lax.fori_loop(0, ngroups, _grp, jnp.int32(0))

            for k in range(NB // L):
                lst[pl.ds(nm + k * L, L)] = jnp.full((L,), GR << 16, jnp.int32)
            _drain((nm + (NB - 1)) // NB)
            plsc.subcore_barrier()

            nrows = jnp.clip(size - (sid * RSH).astype(jnp.int32), 0, RSH)
            nfull = nrows // NB

            def _wblock(rbase, m):
                pltpu.sync_copy(acc.at[pl.ds(rbase, m)],
                                oh.at[pl.ds(lo + rbase, m)])
                pltpu.sync_copy(cnt.at[pl.ds(rbase, m)],
                                ch.at[pl.ds(lo + rbase, m)])

            def _wchunk(cch, _):
                _wblock(r0 + cch * NB, NB)
                return 0
            lax.fori_loop(0, nfull, _wchunk, 0)

            def _wtail(g, _):
                _wblock(r0 + nfull * NB + g * 8, 8)
                return 0
            lax.fori_loop(0, (nrows - nfull * NB) // 8, _wtail, 0)
            plsc.subcore_barrier()


def _sc_aggregate(x_user, x_item, s_click, d_click, s_cb, d_cb):
    ep = s_click.shape[0]
    tile_share = ep // NT
    nchunks = tile_share // ECH
    mesh = plsc.VectorSubcoreMesh(core_axis_name="c", subcore_axis_name="s")
    f32 = jnp.float32
    kern = pl.kernel(
        functools.partial(_sc_body, nchunks, tile_share),
        out_type=(jax.ShapeDtypeStruct((N_NODES, D), f32),
                  jax.ShapeDtypeStruct((N_NODES, L), f32),
                  jax.ShapeDtypeStruct((N_NODES, D), f32),
                  jax.ShapeDtypeStruct((N_NODES, L), f32)),
        mesh=mesh,
        compiler_params=pltpu.CompilerParams(needs_layout_passes=False,
                                             use_tc_tiling_on_sc=False),
        scratch_types=[
            pltpu.VMEM((LCAP,), jnp.int32),
            pltpu.VMEM((2, ECH), jnp.int32),
            pltpu.VMEM((2, ECH), jnp.int32),
            pltpu.VMEM((2, NB, D), f32),
            pltpu.VMEM((NB, L), f32),
            pltpu.VMEM((2, NB), jnp.int32),
            pltpu.VMEM((2, NB), jnp.int32),
            pltpu.VMEM((NB, L), f32),
            pltpu.VMEM_SHARED((CPAD, D), f32),
            pltpu.VMEM_SHARED((CPAD, L), f32),
            pltpu.SemaphoreType.DMA((2,)),
            pltpu.SemaphoreType.DMA((2,)),
            pltpu.SemaphoreType.DMA((2,)),
            pltpu.SemaphoreType.DMA((2,)),
        ],
    )
    return kern(x_user, x_item, s_click, d_click, s_cb, d_cb)


def _mm_body(x_ref, c_ref, w_ref, b_ref, o_ref):
    recip = 1.0 / jnp.maximum(c_ref[...][:, 0:1], 1.0)
    o_ref[...] = jnp.dot(x_ref[...] * recip, w_ref[...],
                         preferred_element_type=jnp.float32) + b_ref[...]


def _transform(sum_x, cnt_x, W, b):
    bm = 2000
    grid = (sum_x.shape[0] // bm,)
    return pl.pallas_call(
        _mm_body,
        grid=grid,
        in_specs=[
            pl.BlockSpec((bm, D), lambda i: (i, 0)),
            pl.BlockSpec((bm, L), lambda i: (i, 0)),
            pl.BlockSpec((D, D), lambda i: (0, 0)),
            pl.BlockSpec((1, D), lambda i: (0, 0)),
        ],
        out_specs=pl.BlockSpec((bm, D), lambda i: (i, 0)),
        out_shape=jax.ShapeDtypeStruct((sum_x.shape[0], D), jnp.float32),
    )(sum_x, cnt_x, W, b.reshape(1, D))


def kernel(x_user, x_item, edge_index_click, edge_index_clicked_by,
           W_click, b_click, W_clicked_by, b_clicked_by):
    e = edge_index_click.shape[1]
    ep = ((e + NT * ECH - 1) // (NT * ECH)) * (NT * ECH)
    pad = ep - e
    sentinel = jnp.int32(1 << 28)

    def prep(ei):
        s = ei[0].astype(jnp.int32)
        d = ei[1].astype(jnp.int32)
        if pad:
            s = jnp.concatenate([s, jnp.zeros((pad,), jnp.int32)])
            d = jnp.concatenate([d, jnp.full((pad,), sentinel, jnp.int32)])
        return s, d

    s_click, d_click = prep(edge_index_click)
    s_cb, d_cb = prep(edge_index_clicked_by)

    sum_click, cnt_click, sum_cb, cnt_cb = _sc_aggregate(
        x_user, x_item, s_click, d_click, s_cb, d_cb)
    h_item = _transform(sum_click, cnt_click, W_click, b_click)
    h_user = _transform(sum_cb, cnt_cb, W_clicked_by, b_clicked_by)
    return (h_user, h_item)


# speedup vs baseline: 1.0217x; 1.0002x over previous
import functools

import jax
import jax.numpy as jnp
from jax import lax
from jax.experimental import pallas as pl
from jax.experimental.pallas import tpu as pltpu
from jax.experimental.pallas import tpu_sc as plsc

N_NODES = 50000
D = 128
L = 16
NT = 16
NSC = 2
N_PASS = 3
CS = 8336
GR = 8400
CPAD = 8448
RSH = CPAD // NT
ECH = 1600
VPC = ECH // L
NB = 128
LCAP = 2 * ECH + 256


def _sc_body(nchunks, tile_share,
             xu, xi, s0, d0, s1, d1, out0, cnt0, out1, cnt1,
             lst, e_s, e_d, rows, ones, srcb, didx, zc,
             acc, cnt, gsem, esem, ssem, csem):
    cid = lax.axis_index("c")
    sid = lax.axis_index("s")

    def _fill(r, _):
        ones[r, :] = jnp.ones((L,), jnp.float32)
        zc[r, :] = jnp.zeros((L,), jnp.float32)
        return 0
    lax.fori_loop(0, NB, _fill, 0)

    for rel in range(2):
        xh = xu if rel == 0 else xi
        sh = s0 if rel == 0 else s1
        dh = d0 if rel == 0 else d1
        oh = out0 if rel == 0 else out1
        ch = cnt0 if rel == 0 else cnt1
        for p in range(N_PASS):
            lo = ((p * NSC + cid) * CS).astype(jnp.int32)
            size = jnp.minimum(N_NODES - lo, CS).astype(jnp.int32)

            def _zrow(r, _):
                for k in range(D // L):
                    rows[0, r, pl.ds(k * L, L)] = jnp.zeros((L,), jnp.float32)
                return 0
            lax.fori_loop(0, NB, _zrow, 0)
            r0 = sid * RSH
            for cch in range(RSH // NB):
                pltpu.sync_copy(rows.at[0], acc.at[pl.ds(r0 + cch * NB, NB)])
                pltpu.sync_copy(zc, cnt.at[pl.ds(r0 + cch * NB, NB)])
            tail = RSH % NB
            if tail:
                base_t = r0 + (RSH // NB) * NB
                pltpu.sync_copy(rows.at[0, pl.ds(0, tail)], acc.at[pl.ds(base_t, tail)])
                pltpu.sync_copy(zc.at[pl.ds(0, tail)], cnt.at[pl.ds(base_t, tail)])
            plsc.subcore_barrier()

            def _fire(j, b):
                base = b * NB
                for k in range(NB // L):
                    pk = lst[pl.ds(base + k * L, L)]
                    didx[j, pl.ds(k * L, L)] = lax.shift_right_logical(pk, 16)
                    srcb[j, pl.ds(k * L, L)] = pk & 0xFFFF
                pltpu.async_copy(xh.at[srcb.at[j]], rows.at[j], gsem.at[j])

            def _waitg(j, b):
                pltpu.make_async_copy(xh.at[srcb.at[j]],
                                      rows.at[j], gsem.at[j]).wait()

            def _scat(j):
                pltpu.async_copy(rows.at[j], acc.at[didx.at[j]],
                                 ssem.at[j], add=True)
                pltpu.async_copy(ones, cnt.at[didx.at[j]],
                                 csem.at[j], add=True)

            def _wait_scat(j):
                pltpu.make_async_copy(rows.at[j], acc.at[didx.at[j]],
                                      ssem.at[j]).wait()
                pltpu.make_async_copy(ones, cnt.at[didx.at[j]],
                                      csem.at[j]).wait()

            def _drain(nb):
                @pl.when(nb > 0)
                def _p():
                    _fire(0, 0)

                def _pair(i, _):
                    b = i * 2

                    @pl.when((b + 1 < nb) & (b >= 2))
                    def _ws1():
                        _wait_scat(1)

                    @pl.when(b + 1 < nb)
                    def _f1():
                        _fire(1, b + 1)
                    _waitg(0, b)
                    _scat(0)

                    @pl.when(b + 2 < nb)
                    def _f0():
                        _wait_scat(0)
                        _fire(0, b + 2)

                    @pl.when(b + 1 < nb)
                    def _w1():
                        _waitg(1, b + 1)
                        _scat(1)
                    return 0
                lax.fori_loop(0, (nb + 1) // 2, _pair, 0)

                @pl.when(nb > 0)
                def _e0():
                    _wait_scat(0)

                @pl.when(nb > 1)
                def _e1():
                    _wait_scat(1)

            def _fire_e(j, c):
                ebase = sid * tile_share + c * ECH
                pltpu.async_copy(sh.at[pl.ds(ebase, ECH)], e_s.at[j],
                                 esem.at[j])
                pltpu.async_copy(dh.at[pl.ds(ebase, ECH)], e_d.at[j],
                                 esem.at[j])

            def _wait_e(j, c):
                ebase = sid * tile_share + c * ECH
                pltpu.make_async_copy(sh.at[pl.ds(ebase, ECH)], e_s.at[j],
                                      esem.at[j]).wait()
                pltpu.make_async_copy(dh.at[pl.ds(ebase, ECH)], e_d.at[j],
                                      esem.at[j]).wait()

            def _scan(j, nm):
                def _vreg4(v4, nm2):
                    for u in range(4):
                        v = v4 * 4 + u
                        sv = e_s[j, pl.ds(v * L, L)]
                        dv = e_d[j, pl.ds(v * L, L)]
                        doff = dv - lo
                        m = (plsc.bitcast(doff, jnp.uint32)
                             < plsc.bitcast(jnp.full((L,), size, jnp.int32),
                                            jnp.uint32))
                        packed = sv | lax.shift_left(doff, 16)
                        plsc.store_compressed(lst.at[pl.ds(nm2, L)], packed,
                                              mask=m)
                        pc = plsc.all_reduce_population_count(m)
                        nm2 = nm2 + pc[0]
                    return nm2
                return lax.fori_loop(0, VPC // 4, _vreg4, nm)

            ngroups = nchunks // 2
            _fire_e(0, 0)

            def _grp(g, nm):
                c0 = g * 2
                _fire_e(1, c0 + 1)
                _wait_e(0, c0)
                nm = _scan(0, nm)

                @pl.when(g < ngroups - 1)
                def _pf():
                    _fire_e(0, c0 + 2)
                _wait_e(1, c0 + 1)
                nm = _scan(1, nm)

                nb = nm // NB
                _drain(nb)
                rebase = nb * NB
                for k in range(NB // L):
                    pkv = lst[pl.ds(rebase + k * L, L)]
                    lst[pl.ds(k * L, L)] = pkv
                return nm - rebase
            nm = lax.fori_loop(0, ngroups, _grp, jnp.int32(0))

            for k in range(NB // L):
                lst[pl.ds(nm + k * L, L)] = jnp.full((L,), GR << 16, jnp.int32)
            _drain((nm + (NB - 1)) // NB)
            plsc.subcore_barrier()

            nrows = jnp.clip(size - (sid * RSH).astype(jnp.int32), 0, RSH)
            nfull = nrows // NB

            def _wblock(rbase, m):
                pltpu.sync_copy(acc.at[pl.ds(rbase, m)],
                                oh.at[pl.ds(lo + rbase, m)])
                pltpu.sync_copy(cnt.at[pl.ds(rbase, m)],
                                ch.at[pl.ds(lo + rbase, m)])

            def _wchunk(cch, _):
                _wblock(r0 + cch * NB, NB)
                return 0
            lax.fori_loop(0, nfull, _wchunk, 0)

            def _wtail(g, _):
                _wblock(r0 + nfull * NB + g * 8, 8)
                return 0
            lax.fori_loop(0, (nrows - nfull * NB) // 8, _wtail, 0)
            plsc.subcore_barrier()


def _sc_aggregate(x_user, x_item, s_click, d_click, s_cb, d_cb):
    ep = s_click.shape[0]
    tile_share = ep // NT
    nchunks = tile_share // ECH
    mesh = plsc.VectorSubcoreMesh(core_axis_name="c", subcore_axis_name="s")
    f32 = jnp.float32
    kern = pl.kernel(
        functools.partial(_sc_body, nchunks, tile_share),
        out_type=(jax.ShapeDtypeStruct((N_NODES, D), f32),
                  jax.ShapeDtypeStruct((N_NODES, L), f32),
                  jax.ShapeDtypeStruct((N_NODES, D), f32),
                  jax.ShapeDtypeStruct((N_NODES, L), f32)),
        mesh=mesh,
        compiler_params=pltpu.CompilerParams(needs_layout_passes=False,
                                             use_tc_tiling_on_sc=False),
        scratch_types=[
            pltpu.VMEM((LCAP,), jnp.int32),
            pltpu.VMEM((2, ECH), jnp.int32),
            pltpu.VMEM((2, ECH), jnp.int32),
            pltpu.VMEM((2, NB, D), f32),
            pltpu.VMEM((NB, L), f32),
            pltpu.VMEM((2, NB), jnp.int32),
            pltpu.VMEM((2, NB), jnp.int32),
            pltpu.VMEM((NB, L), f32),
            pltpu.VMEM_SHARED((CPAD, D), f32),
            pltpu.VMEM_SHARED((CPAD, L), f32),
            pltpu.SemaphoreType.DMA((2,)),
            pltpu.SemaphoreType.DMA((2,)),
            pltpu.SemaphoreType.DMA((2,)),
            pltpu.SemaphoreType.DMA((2,)),
        ],
    )
    return kern(x_user, x_item, s_click, d_click, s_cb, d_cb)


def _mm_body(x_ref, c_ref, w_ref, b_ref, o_ref):
    recip = 1.0 / jnp.maximum(c_ref[...][:, 0:1], 1.0)
    o_ref[...] = jnp.dot(x_ref[...] * recip, w_ref[...],
                         preferred_element_type=jnp.float32) + b_ref[...]


def _transform(sum_x, cnt_x, W, b):
    bm = 2000
    grid = (sum_x.shape[0] // bm,)
    return pl.pallas_call(
        _mm_body,
        grid=grid,
        in_specs=[
            pl.BlockSpec((bm, D), lambda i: (i, 0)),
            pl.BlockSpec((bm, L), lambda i: (i, 0)),
            pl.BlockSpec((D, D), lambda i: (0, 0)),
            pl.BlockSpec((1, D), lambda i: (0, 0)),
        ],
        out_specs=pl.BlockSpec((bm, D), lambda i: (i, 0)),
        out_shape=jax.ShapeDtypeStruct((sum_x.shape[0], D), jnp.float32),
    )(sum_x, cnt_x, W, b.reshape(1, D))


def kernel(x_user, x_item, edge_index_click, edge_index_clicked_by,
           W_click, b_click, W_clicked_by, b_clicked_by):
    e = edge_index_click.shape[1]
    ep = ((e + NT * ECH - 1) // (NT * ECH)) * (NT * ECH)
    pad = ep - e
    sentinel = jnp.int32(1 << 28)

    def prep(ei):
        s = ei[0].astype(jnp.int32)
        d = ei[1].astype(jnp.int32)
        if pad:
            s = jnp.concatenate([s, jnp.zeros((pad,), jnp.int32)])
            d = jnp.concatenate([d, jnp.full((pad,), sentinel, jnp.int32)])
        return s, d

    s_click, d_click = prep(edge_index_click)
    s_cb, d_cb = prep(edge_index_clicked_by)

    sum_click, cnt_click, sum_cb, cnt_cb = _sc_aggregate(
        x_user, x_item, s_click, d_click, s_cb, d_cb)
    h_item = _transform(sum_click, cnt_click, W_click, b_click)
    h_user = _transform(sum_cb, cnt_cb, W_clicked_by, b_clicked_by)
    return (h_user, h_item)
